# baseline (speedup 1.0000x reference)
import functools

import jax
import jax.numpy as jnp
from jax import lax
from jax.experimental import pallas as pl
from jax.experimental.pallas import tpu as pltpu
from jax.experimental.pallas import tpu_sc as plsc

N_CB = 14
VOCAB = 1024
D = 1024
MASK = 1024
ROWS = VOCAB + 1
BATCH = 4
SEQ = 2048
NPOS = BATCH * SEQ
BV = 128
CH = 8
K_TC = 7
N_A = N_CB - K_TC
PB = 1024
AB = 512


def _proj_body(t_ref, me_ref, w_ref, b_ref, p_ref, *, cb0):
    i = pl.program_id(0)
    t = t_ref[0]
    rows = lax.broadcasted_iota(jnp.int32, (ROWS, 1), 0)
    t = jnp.where(rows == MASK, me_ref[0], t)
    acc = lax.dot_general(t, w_ref[...], (((1,), (1,)), ((), ())),
                          preferred_element_type=jnp.float32)
    p_ref[0] = acc + jnp.where(i + cb0 == 0, b_ref[...], 0.0)


def _project(tables, mask_emb, W, b2d, cb0, ncb):
    return pl.pallas_call(
        functools.partial(_proj_body, cb0=cb0),
        grid=(ncb,),
        in_specs=[
            pl.BlockSpec((1, ROWS, D), lambda i: (i + cb0, 0, 0)),
            pl.BlockSpec((1, 1, D), lambda i: (i + cb0, 0, 0)),
            pl.BlockSpec((D, D), lambda i: (0, i + cb0)),
            pl.BlockSpec((1, D), lambda i: (0, 0)),
        ],
        out_specs=pl.BlockSpec((1, ROWS, D), lambda i: (i, 0, 0)),
        out_shape=jax.ShapeDtypeStruct((ncb, ROWS, D), jnp.float32),
    )(tables, mask_emb.reshape(N_CB, 1, D), W, b2d)


def _sc_body(codes_hbm, p_hbm, out_hbm, idx_v, gbuf, accv,
             gsem0, gsem1, osem):
    info = plsc.get_sparse_core_info()
    nc = info.num_cores
    wid = lax.axis_index("s") * nc + lax.axis_index("c")
    per_w = NPOS // (nc * info.num_subcores)
    base = wid * per_w
    b_idx = base // SEQ
    s0 = base % SEQ

    cdescs = []
    for k in range(N_A):
        i = K_TC + k
        cdescs.append(pltpu.async_copy(
            codes_hbm.at[b_idx * N_CB + i, pl.ds(s0, per_w)],
            idx_v.at[k], osem))
    for d in cdescs:
        d.wait()
    for k in range(N_A):
        off = k * ROWS

        def _off_body(j, _, k=k, off=off):
            idx_v[k, pl.ds(j * 16, 16)] = idx_v[k, pl.ds(j * 16, 16)] + off
            return 0
        lax.fori_loop(0, per_w // 16, _off_body, 0)

    nchunks = per_w // CH
    ga = (N_A + 1) // 2
    gsizes = (ga, N_A - ga)
    gbases = (0, ga)
    gsems = (gsem0, gsem1)

    def _fire(j, g):
        for k in range(gsizes[g]):
            pltpu.async_copy(
                p_hbm.at[idx_v.at[gbases[g] + k, pl.ds(j * CH, CH)]],
                gbuf.at[g, k], gsems[g])

    def _drain_gather(g):
        for k in range(gsizes[g]):
            pltpu.make_async_copy(p_hbm.at[pl.ds(0, CH)], gbuf.at[g, k],
                                  gsems[g]).wait()

    def _drain_out():
        pltpu.make_async_copy(accv, out_hbm.at[pl.ds(base, CH)], osem).wait()

    _fire(0, 0)
    _fire(0, 1)

    def _chunk(j, _):
        for g in range(2):
            _drain_gather(g)
            if g == 0:
                @pl.when(j > 0)
                def _():
                    _drain_out()
            for p in range(CH):
                def _col(v, _, p=p, g=g):
                    sl = pl.ds(v * 16, 16)
                    acc = gbuf[g, 0, p, sl] if g == 0 else accv[p, sl]
                    for k in range(0 if g else 1, gsizes[g]):
                        acc = acc + gbuf[g, k, p, sl]
                    accv[p, sl] = acc
                    return 0
                lax.fori_loop(0, D // 16, _col, 0)

            @pl.when(j + 1 < nchunks)
            def _(g=g):
                _fire(j + 1, g)
        pltpu.async_copy(accv, out_hbm.at[pl.ds(base + j * CH, CH)], osem)
        return 0

    lax.fori_loop(0, nchunks, _chunk, 0)
    _drain_out()


def _gather_sum(codes2, p_flat):
    mesh = plsc.VectorSubcoreMesh(core_axis_name="c", subcore_axis_name="s")
    f = functools.partial(
        pl.kernel,
        mesh=mesh,
        out_type=jax.ShapeDtypeStruct((NPOS, D), jnp.float32),
        scratch_types=[
            pltpu.VMEM((N_A + (N_A % 8), 256), jnp.int32),
            pltpu.VMEM((2, (N_A + 1) // 2, CH, D), jnp.float32),
            pltpu.VMEM((CH, D), jnp.float32),
            pltpu.SemaphoreType.DMA,
            pltpu.SemaphoreType.DMA,
            pltpu.SemaphoreType.DMA,
        ],
    )(_sc_body)
    return f(codes2, p_flat)


def _oh_body(ct_ref, p_ref, o_ref):
    i = pl.program_id(1)
    cb = ct_ref[0]
    col = jnp.reshape(cb, (PB, 1))
    oh = (col == lax.broadcasted_iota(jnp.int32, (PB, ROWS), 1))
    oh = oh.astype(jnp.float32)
    prod = lax.dot_general(oh, p_ref[0], (((1,), (0,)), ((), ())),
                           preferred_element_type=jnp.float32)

    @pl.when(i == 0)
    def _():
        o_ref[...] = prod

    @pl.when(i > 0)
    def _():
        o_ref[...] = o_ref[...] + prod


def _onehot_part(codes_tc, p_full):
    return pl.pallas_call(
        _oh_body,
        grid=(NPOS // PB, K_TC),
        in_specs=[
            pl.BlockSpec((1, 1, PB), lambda pb, i: (i, 0, pb)),
            pl.BlockSpec((1, ROWS, D), lambda pb, i: (i, 0, 0)),
        ],
        out_specs=pl.BlockSpec((PB, D), lambda pb, i: (pb, 0)),
        out_shape=jax.ShapeDtypeStruct((NPOS, D), jnp.float32),
    )(codes_tc, p_full)


def _add_body(a_ref, b_ref, o_ref):
    o_ref[...] = a_ref[...] + b_ref[...]


def _combine(pa, pb):
    return pl.pallas_call(
        _add_body,
        grid=(NPOS // AB,),
        in_specs=[pl.BlockSpec((AB, D), lambda pb: (pb, 0)),
                  pl.BlockSpec((AB, D), lambda pb: (pb, 0))],
        out_specs=pl.BlockSpec((AB, D), lambda pb: (pb, 0)),
        out_shape=jax.ShapeDtypeStruct((NPOS, D), jnp.float32),
    )(pa, pb)


def kernel(codes, tables, mask_emb, W, b):
    b2d = b.reshape(1, D)
    p_a = _project(tables, mask_emb, W, b2d, K_TC, N_A)
    p_b = _project(tables, mask_emb, W, b2d, 0, K_TC)
    codes_i32 = codes.astype(jnp.int32)
    codes2 = codes_i32.reshape(BATCH * N_CB, SEQ)
    codes_tc = codes_i32.transpose(1, 0, 2).reshape(N_CB, 1, NPOS)[:K_TC]
    part_sc = _gather_sum(codes2, p_a.reshape(N_A * ROWS, D))
    part_tc = _onehot_part(codes_tc, p_b)
    out = _combine(part_sc, part_tc)
    return out.reshape(BATCH, SEQ, D)


# speedup vs baseline: 1.0158x; 1.0158x over previous
import functools

import jax
import jax.numpy as jnp
from jax import lax
from jax.experimental import pallas as pl
from jax.experimental.pallas import tpu as pltpu
from jax.experimental.pallas import tpu_sc as plsc

N_CB = 14
VOCAB = 1024
D = 1024
MASK = 1024
ROWS = VOCAB + 1
BATCH = 4
SEQ = 2048
NPOS = BATCH * SEQ
BV = 128
CH = 8
K_TC = 6
N_A = N_CB - K_TC
PB = 1024
AB = 512


def _proj_body(t_ref, me_ref, w_ref, b_ref, p_ref, *, cb0):
    i = pl.program_id(0)
    t = t_ref[0]
    rows = lax.broadcasted_iota(jnp.int32, (ROWS, 1), 0)
    t = jnp.where(rows == MASK, me_ref[0], t)
    acc = lax.dot_general(t, w_ref[...], (((1,), (1,)), ((), ())),
                          preferred_element_type=jnp.float32)
    p_ref[0] = acc + jnp.where(i + cb0 == 0, b_ref[...], 0.0)


def _project(tables, mask_emb, W, b2d, cb0, ncb):
    return pl.pallas_call(
        functools.partial(_proj_body, cb0=cb0),
        grid=(ncb,),
        in_specs=[
            pl.BlockSpec((1, ROWS, D), lambda i: (i + cb0, 0, 0)),
            pl.BlockSpec((1, 1, D), lambda i: (i + cb0, 0, 0)),
            pl.BlockSpec((D, D), lambda i: (0, i + cb0)),
            pl.BlockSpec((1, D), lambda i: (0, 0)),
        ],
        out_specs=pl.BlockSpec((1, ROWS, D), lambda i: (i, 0, 0)),
        out_shape=jax.ShapeDtypeStruct((ncb, ROWS, D), jnp.float32),
    )(tables, mask_emb.reshape(N_CB, 1, D), W, b2d)


def _sc_body(codes_hbm, p_hbm, out_hbm, idx_v, gbuf, accv,
             gsem0, gsem1, osem):
    info = plsc.get_sparse_core_info()
    nc = info.num_cores
    wid = lax.axis_index("s") * nc + lax.axis_index("c")
    per_w = NPOS // (nc * info.num_subcores)
    base = wid * per_w
    b_idx = base // SEQ
    s0 = base % SEQ

    cdescs = []
    for k in range(N_A):
        i = K_TC + k
        cdescs.append(pltpu.async_copy(
            codes_hbm.at[b_idx * N_CB + i, pl.ds(s0, per_w)],
            idx_v.at[k], osem))
    for d in cdescs:
        d.wait()
    for k in range(N_A):
        off = k * ROWS

        def _off_body(j, _, k=k, off=off):
            idx_v[k, pl.ds(j * 16, 16)] = idx_v[k, pl.ds(j * 16, 16)] + off
            return 0
        lax.fori_loop(0, per_w // 16, _off_body, 0)

    nchunks = per_w // CH
    ga = (N_A + 1) // 2
    gsizes = (ga, N_A - ga)
    gbases = (0, ga)
    gsems = (gsem0, gsem1)

    def _fire(j, g):
        for k in range(gsizes[g]):
            pltpu.async_copy(
                p_hbm.at[idx_v.at[gbases[g] + k, pl.ds(j * CH, CH)]],
                gbuf.at[g, k], gsems[g])

    def _drain_gather(g):
        for k in range(gsizes[g]):
            pltpu.make_async_copy(p_hbm.at[pl.ds(0, CH)], gbuf.at[g, k],
                                  gsems[g]).wait()

    def _drain_out():
        pltpu.make_async_copy(accv, out_hbm.at[pl.ds(base, CH)], osem).wait()

    _fire(0, 0)
    _fire(0, 1)

    def _chunk(j, _):
        for g in range(2):
            _drain_gather(g)
            if g == 0:
                @pl.when(j > 0)
                def _():
                    _drain_out()
            for p in range(CH):
                def _col(v, _, p=p, g=g):
                    sl = pl.ds(v * 16, 16)
                    acc = gbuf[g, 0, p, sl] if g == 0 else accv[p, sl]
                    for k in range(0 if g else 1, gsizes[g]):
                        acc = acc + gbuf[g, k, p, sl]
                    accv[p, sl] = acc
                    return 0
                lax.fori_loop(0, D // 16, _col, 0)

            @pl.when(j + 1 < nchunks)
            def _(g=g):
                _fire(j + 1, g)
        pltpu.async_copy(accv, out_hbm.at[pl.ds(base + j * CH, CH)], osem)
        return 0

    lax.fori_loop(0, nchunks, _chunk, 0)
    _drain_out()


def _gather_sum(codes2, p_flat):
    mesh = plsc.VectorSubcoreMesh(core_axis_name="c", subcore_axis_name="s")
    f = functools.partial(
        pl.kernel,
        mesh=mesh,
        out_type=jax.ShapeDtypeStruct((NPOS, D), jnp.float32),
        scratch_types=[
            pltpu.VMEM((N_A + (N_A % 8), 256), jnp.int32),
            pltpu.VMEM((2, (N_A + 1) // 2, CH, D), jnp.float32),
            pltpu.VMEM((CH, D), jnp.float32),
            pltpu.SemaphoreType.DMA,
            pltpu.SemaphoreType.DMA,
            pltpu.SemaphoreType.DMA,
        ],
    )(_sc_body)
    return f(codes2, p_flat)


def _oh_body(ct_ref, p_ref, o_ref):
    i = pl.program_id(1)
    cb = ct_ref[0]
    col = jnp.reshape(cb, (PB, 1))
    oh = (col == lax.broadcasted_iota(jnp.int32, (PB, ROWS), 1))
    oh = oh.astype(jnp.float32)
    prod = lax.dot_general(oh, p_ref[0], (((1,), (0,)), ((), ())),
                           preferred_element_type=jnp.float32)

    @pl.when(i == 0)
    def _():
        o_ref[...] = prod

    @pl.when(i > 0)
    def _():
        o_ref[...] = o_ref[...] + prod


def _onehot_part(codes_tc, p_full):
    return pl.pallas_call(
        _oh_body,
        grid=(NPOS // PB, K_TC),
        in_specs=[
            pl.BlockSpec((1, 1, PB), lambda pb, i: (i, 0, pb)),
            pl.BlockSpec((1, ROWS, D), lambda pb, i: (i, 0, 0)),
        ],
        out_specs=pl.BlockSpec((PB, D), lambda pb, i: (pb, 0)),
        out_shape=jax.ShapeDtypeStruct((NPOS, D), jnp.float32),
    )(codes_tc, p_full)


def _add_body(a_ref, b_ref, o_ref):
    o_ref[...] = a_ref[...] + b_ref[...]


def _combine(pa, pb):
    return pl.pallas_call(
        _add_body,
        grid=(NPOS // AB,),
        in_specs=[pl.BlockSpec((AB, D), lambda pb: (pb, 0)),
                  pl.BlockSpec((AB, D), lambda pb: (pb, 0))],
        out_specs=pl.BlockSpec((AB, D), lambda pb: (pb, 0)),
        out_shape=jax.ShapeDtypeStruct((NPOS, D), jnp.float32),
    )(pa, pb)


def kernel(codes, tables, mask_emb, W, b):
    b2d = b.reshape(1, D)
    p_a = _project(tables, mask_emb, W, b2d, K_TC, N_A)
    p_b = _project(tables, mask_emb, W, b2d, 0, K_TC)
    codes_i32 = codes.astype(jnp.int32)
    codes2 = codes_i32.reshape(BATCH * N_CB, SEQ)
    codes_tc = codes_i32.transpose(1, 0, 2).reshape(N_CB, 1, NPOS)[:K_TC]
    part_sc = _gather_sum(codes2, p_a.reshape(N_A * ROWS, D))
    part_tc = _onehot_part(codes_tc, p_b)
    out = _combine(part_sc, part_tc)
    return out.reshape(BATCH, SEQ, D)
